# baseline (speedup 1.0000x reference)
import jax
import jax.numpy as jnp
from jax import lax
from jax.experimental import pallas as pl
from jax.experimental.pallas import tpu as pltpu
from jax.experimental.pallas import tpu_sc as plsc

_NC = 2
_NS = 16
_NW = _NC * _NS
_K = 8


def _make_sc_kernel(B, L, P, D, K):
    rows_per_w = P // _NW
    mesh = plsc.VectorSubcoreMesh(core_axis_name="c", subcore_axis_name="s")

    def body(input_ids_hbm, prompt_hbm, idx_hbm, table_hbm,
             ids_out_hbm, emb_out_hbm,
             idx_v, rows_v, ids_v, gsem, wsem):
        wid = lax.axis_index("s") * _NC + lax.axis_index("c")
        base = wid * rows_per_w
        pltpu.sync_copy(idx_hbm.at[pl.ds(base, rows_per_w)], idx_v)
        gather = pltpu.async_copy(table_hbm.at[idx_v], rows_v, gsem)

        @pl.when(wid < B)
        def _():
            pltpu.sync_copy(prompt_hbm, ids_v)
            pltpu.sync_copy(ids_v, ids_out_hbm.at[wid, pl.ds(0, P)])
            pltpu.sync_copy(input_ids_hbm.at[wid], ids_v)
            pltpu.sync_copy(ids_v, ids_out_hbm.at[wid, pl.ds(P, L)])

        gather.wait()
        writes = [
            pltpu.async_copy(
                rows_v, emb_out_hbm.at[b, pl.ds(base, rows_per_w)], wsem)
            for b in range(K)
        ]
        for w in writes:
            w.wait()

    return pl.kernel(
        body,
        out_type=(
            jax.ShapeDtypeStruct((B, P + L), jnp.int32),
            jax.ShapeDtypeStruct((K, P, D), jnp.float32),
        ),
        mesh=mesh,
        scratch_types=[
            pltpu.VMEM((rows_per_w,), jnp.int32),
            pltpu.VMEM((rows_per_w, D), jnp.float32),
            pltpu.VMEM((max(P, L),), jnp.int32),
            pltpu.SemaphoreType.DMA,
            pltpu.SemaphoreType.DMA,
        ],
    )


def _tc_broadcast(idx, table_aug, NB, P, D):
    RB = 16
    RPB = P // RB

    def body(idx_smem, table_vmem, out_vmem, scratch_vmem):
        rb = pl.program_id(0)
        b = pl.program_id(1)

        @pl.when(b == 0)
        def _():
            def gather_row(i, c):
                scratch_vmem[i, :] = table_vmem[idx_smem[rb * RPB + i], :]
                return c
            lax.fori_loop(0, RPB, gather_row, 0, unroll=8)

        out_vmem[0] = scratch_vmem[...]

    return pl.pallas_call(
        body,
        grid=(RB, NB),
        in_specs=[
            pl.BlockSpec(memory_space=pltpu.SMEM),
            pl.BlockSpec((table_aug.shape[0], D), lambda rb, b: (0, 0)),
        ],
        out_specs=pl.BlockSpec((1, RPB, D), lambda rb, b: (b, rb, 0)),
        out_shape=jax.ShapeDtypeStruct((NB, P, D), jnp.float32),
        scratch_shapes=[pltpu.VMEM((RPB, D), jnp.float32)],
    )(idx, table_aug)


def kernel(input_ids, prompt_ids, embed_weight):
    B, L = input_ids.shape
    P, D = embed_weight.shape
    idx = jnp.where(prompt_ids < P, prompt_ids, P).astype(jnp.int32)
    table_aug = jnp.concatenate(
        [embed_weight, jnp.full((8, D), jnp.nan, embed_weight.dtype)], axis=0)
    sc = _make_sc_kernel(B, L, P, D, _K)
    prepended_ids, emb_lo = sc(input_ids, prompt_ids, idx, table_aug)
    emb_hi = _tc_broadcast(idx, table_aug, B - _K, P, D)
    prompt_embeds = jnp.concatenate([emb_lo, emb_hi], axis=0)
    return prepended_ids, prompt_embeds
